# baseline (speedup 1.0000x reference)
import functools

import jax
import jax.numpy as jnp
from jax import lax
from jax.experimental import pallas as pl
from jax.experimental.pallas import tpu as pltpu
from jax.experimental.pallas import tpu_sc as plsc

N = 16384
C = 26
CARD = 100
D = C * CARD
NW = 32
SAMPLES_PER_W = N // NW
FB = 2 * CARD
SB = 128
NH = SAMPLES_PER_W // SB
NSUB = SB // 16
NCHUNK = (C // 2) * NH
NBUF = 4

_OUT_DTYPE = jax.dtypes.canonicalize_dtype(jnp.int64)

_mesh = plsc.VectorSubcoreMesh(core_axis_name="c", subcore_axis_name="s")


@functools.partial(
    pl.kernel,
    mesh=_mesh,
    compiler_params=pltpu.CompilerParams(needs_layout_passes=False),
    out_type=jax.ShapeDtypeStruct((D, N), _OUT_DTYPE),
    scratch_types=[
        pltpu.VMEM((C, SAMPLES_PER_W), jnp.int32),
        pltpu.VMEM((FB, SB), _OUT_DTYPE),
        pltpu.VMEM((FB, SB), _OUT_DTYPE),
        pltpu.VMEM((FB, SB), _OUT_DTYPE),
        pltpu.VMEM((FB, SB), _OUT_DTYPE),
        pltpu.SemaphoreType.DMA,
        pltpu.SemaphoreType.DMA,
        pltpu.SemaphoreType.DMA,
        pltpu.SemaphoreType.DMA,
        pltpu.SemaphoreType.DMA,
    ],
)
def _one_hot_sc(xt_hbm, out_hbm, xv, buf0, buf1, buf2, buf3,
                sem0, sem1, sem2, sem3, semx):
    wid = lax.axis_index("s") * 2 + lax.axis_index("c")
    base = wid * SAMPLES_PER_W

    pltpu.async_copy(xt_hbm.at[:, pl.ds(base, SAMPLES_PER_W)], xv, semx)

    lanes = lax.iota(jnp.int32, 16)
    ones = jnp.full((16,), 1, _OUT_DTYPE)
    zeros = jnp.zeros((16,), _OUT_DTYPE)
    zvec = jnp.zeros((16,), _OUT_DTYPE)

    bufs = (buf0, buf1, buf2, buf3)
    sems = (sem0, sem1, sem2, sem3)

    def _memset(buf):
        def _zero(r, _):
            for k in range(NSUB):
                buf[r, pl.ds(k * 16, 16)] = zvec
            return _
        lax.fori_loop(0, FB, _zero, 0)

    _memset(bufs[0])
    pltpu.make_async_copy(
        xt_hbm.at[:, pl.ds(base, SAMPLES_PER_W)], xv, semx).wait()

    def _scatter_chunk(buf, q, data):
        c2 = q // NH
        h = q - c2 * NH
        for c_off in range(2):
            c = 2 * c2 + c_off
            for k in range(NSUB):
                s = k * 16
                vals = xv[c, pl.ds(h * SB + s, 16)]
                feat = vals + (c_off * CARD)
                col = lanes + s
                plsc.store_scatter(buf, [feat, col], data)

    def _dma_out(buf, sem, q):
        c2 = q // NH
        h = q - c2 * NH
        dst = out_hbm.at[pl.ds(c2 * FB, FB), pl.ds(base + h * SB, SB)]
        pltpu.async_copy(buf, dst, sem)

    def _dma_wait(buf, sem):
        dst = out_hbm.at[pl.ds(0, FB), pl.ds(base, SB)]
        pltpu.make_async_copy(buf, dst, sem).wait()

    for b in range(NBUF):
        if b > 0:
            _memset(bufs[b])
        _scatter_chunk(bufs[b], b, ones)
        _dma_out(bufs[b], sems[b], b)

    def _body(i, _):
        for b in range(NBUF):
            q = i * NBUF + NBUF + b
            _dma_wait(bufs[b], sems[b])
            _scatter_chunk(bufs[b], q - NBUF, zeros)
            _scatter_chunk(bufs[b], q, ones)
            _dma_out(bufs[b], sems[b], q)
        return _
    lax.fori_loop(0, (NCHUNK - NBUF) // NBUF, _body, 0)

    for b in range(NBUF):
        _dma_wait(bufs[b], sems[b])


def kernel(x):
    out_t = _one_hot_sc(jnp.asarray(x, jnp.int32).T)
    return out_t.T
